# baseline (speedup 1.0000x reference)
import functools

import jax
import jax.numpy as jnp
from jax import lax
from jax.experimental import pallas as pl
from jax.experimental.pallas import tpu as pltpu
from jax.experimental.pallas import tpu_sc as plsc

_B, _S, _ML, _D, _F = 16, 512, 2048, 256, 256
_NROWS = _B * _ML
_ZROW = _B * _S
_NW = 32
_RPW = _NROWS // _NW
_CH = 64
_NCH = _RPW // _CH


def _conv3(xin, w_ref, b_ref):
    zero = jnp.zeros((1, xin.shape[1]), jnp.float32)
    xprev = jnp.concatenate([zero, xin[:-1]], axis=0)
    xnext = jnp.concatenate([xin[1:], zero], axis=0)
    y = jnp.dot(xprev, w_ref[0], preferred_element_type=jnp.float32)
    y = y + jnp.dot(xin, w_ref[1], preferred_element_type=jnp.float32)
    y = y + jnp.dot(xnext, w_ref[2], preferred_element_type=jnp.float32)
    return y + b_ref[:, :]


def _ln(h, g_ref, be_ref):
    m = jnp.mean(h, axis=-1, keepdims=True)
    d = h - m
    v = jnp.mean(d * d, axis=-1, keepdims=True)
    return d * jax.lax.rsqrt(v + 1e-5) * g_ref[:, :] + be_ref[:, :]


def _predictor(xin, w1, b1, g1, be1, w2, b2, g2, be2, lwt, lb):
    h = jnp.maximum(_conv3(xin, w1, b1), 0.0)
    h = _ln(h, g1, be1)
    h = jnp.maximum(_conv3(h, w2, b2), 0.0)
    h = _ln(h, g2, be2)
    out = jnp.sum(h * lwt[:, :], axis=-1, keepdims=True)
    return out + lb[:, :]


def _pack_params(p):
    return [
        p['w1'], p['b1'].reshape(1, _F), p['g1'].reshape(1, _F), p['be1'].reshape(1, _F),
        p['w2'], p['b2'].reshape(1, _F), p['g2'].reshape(1, _F), p['be2'].reshape(1, _F),
        p['lw'].reshape(1, _F), p['lb'].reshape(1, 1),
    ]


def _full_spec(arr):
    return pl.BlockSpec(arr.shape, lambda i: (0,) * arr.ndim)


def _front_body(xf_ref, dur_ref, pt_ref, et_ref, pbins_ref, ebins_ref,
                dw1, db1, dg1, dbe1, dw2, db2, dg2, dbe2, dlwt, dlb,
                logdur_ref, mel_ref, gidx_ref, pidx_ref, eidx_ref):
    b = pl.program_id(0)
    xf_b = xf_ref[0]
    dur_row = dur_ref[0].astype(jnp.float32)

    jj = lax.broadcasted_iota(jnp.int32, (_S, _S), 0)
    ss = lax.broadcasted_iota(jnp.int32, (_S, _S), 1)
    tri = (jj <= ss).astype(jnp.float32)
    cum_row = jnp.dot(dur_row, tri, preferred_element_type=jnp.float32)

    t_col = lax.broadcasted_iota(jnp.int32, (_ML, 1), 0).astype(jnp.float32)
    src = jnp.sum((cum_row <= t_col).astype(jnp.float32), axis=-1, keepdims=True)
    mel_f = jnp.sum(dur_row, axis=-1, keepdims=True)
    valid = t_col < jnp.minimum(mel_f, jnp.float32(_ML))
    base_f = (b * _S).astype(jnp.float32)
    gidx_f = jnp.where(valid, src + base_f, jnp.float32(_ZROW))
    gidx_ref[0] = gidx_f.astype(jnp.int32)

    p_col = pt_ref[0]
    e_col = et_ref[0]
    pidx = jnp.sum((pbins_ref[:, :] < p_col).astype(jnp.float32), axis=-1, keepdims=True)
    eidx = jnp.sum((ebins_ref[:, :] < e_col).astype(jnp.float32), axis=-1, keepdims=True)
    pidx_ref[0] = pidx.astype(jnp.int32)
    eidx_ref[0] = eidx.astype(jnp.int32)

    logdur_ref[0] = _predictor(xf_b, dw1, db1, dg1, dbe1, dw2, db2, dg2, dbe2, dlwt, dlb)
    mel_ref[0] = cum_row[:, _S - 128:]


def _sc_body(xpad_hbm, xfpad_hbm, pemb_hbm, eemb_hbm, gidx_hbm, pidx_hbm, eidx_hbm,
             xout_hbm, xfexp_hbm,
             gix_v, pix_v, eix_v, xbuf, xfbuf, pebuf, eebuf,
             sem1, sem2, sem3, sem4):
    wid = lax.axis_index("s") * 2 + lax.axis_index("c")

    def chunk(k, carry):
        base = wid * _RPW + k * _CH
        pltpu.sync_copy(gidx_hbm.at[pl.ds(base, _CH)], gix_v)
        pltpu.sync_copy(pidx_hbm.at[pl.ds(base, _CH)], pix_v)
        pltpu.sync_copy(eidx_hbm.at[pl.ds(base, _CH)], eix_v)
        cp_x = pltpu.async_copy(xpad_hbm.at[gix_v], xbuf, sem1)
        cp_xf = pltpu.async_copy(xfpad_hbm.at[gix_v], xfbuf, sem2)
        cp_pe = pltpu.async_copy(pemb_hbm.at[pix_v], pebuf, sem3)
        cp_ee = pltpu.async_copy(eemb_hbm.at[eix_v], eebuf, sem4)
        cp_xf.wait()
        pltpu.sync_copy(xfbuf, xfexp_hbm.at[pl.ds(base, _CH)])
        cp_x.wait()
        cp_pe.wait()
        cp_ee.wait()

        def row(r, c):
            for j in range(_D // 16):
                sl = pl.ds(j * 16, 16)
                xbuf[r, sl] = xbuf[r, sl] + pebuf[r, sl] + eebuf[r, sl]
            return c

        lax.fori_loop(0, _CH, row, 0)
        pltpu.sync_copy(xbuf, xout_hbm.at[pl.ds(base, _CH)])
        return carry

    lax.fori_loop(0, _NCH, chunk, 0)


def _back_body(xfexp_ref,
               pw1, pb1, pg1, pbe1, pw2, pb2, pg2, pbe2, plwt, plb,
               ew1, eb1, eg1, ebe1, ew2, eb2, eg2, ebe2, elwt, elb,
               ppred_ref, epred_ref):
    xf_exp = xfexp_ref[0]
    ppred_ref[0] = _predictor(xf_exp, pw1, pb1, pg1, pbe1, pw2, pb2, pg2, pbe2, plwt, plb)
    epred_ref[0] = _predictor(xf_exp, ew1, eb1, eg1, ebe1, ew2, eb2, eg2, ebe2, elwt, elb)


def kernel(x, x_features, src_mask, mel_mask, duration_target, pitch_target,
           energy_target, max_len, dur_params, pitch_params, energy_params,
           pitch_bins, energy_bins, pitch_embedding, energy_embedding):
    B, S, D = x.shape
    ML = mel_mask.shape[1]

    dur3 = duration_target.reshape(B, 1, S)
    pt3 = pitch_target.reshape(B, ML, 1)
    et3 = energy_target.reshape(B, ML, 1)
    pad = jnp.full((1,), jnp.inf, jnp.float32)
    pbins = jnp.concatenate([pitch_bins, pad]).reshape(1, 256)
    ebins = jnp.concatenate([energy_bins, pad]).reshape(1, 256)

    batch3 = lambda i: (i, 0, 0)
    dparams = _pack_params(dur_params)
    pparams = _pack_params(pitch_params)
    eparams = _pack_params(energy_params)

    front_in_specs = [
        pl.BlockSpec((1, S, D), batch3),
        pl.BlockSpec((1, 1, S), batch3),
        pl.BlockSpec((1, ML, 1), batch3),
        pl.BlockSpec((1, ML, 1), batch3),
        _full_spec(pbins), _full_spec(ebins),
    ] + [_full_spec(a) for a in dparams]
    front_out_shapes = [
        jax.ShapeDtypeStruct((B, S, 1), jnp.float32),
        jax.ShapeDtypeStruct((B, 1, 128), jnp.float32),
        jax.ShapeDtypeStruct((B, ML, 1), jnp.int32),
        jax.ShapeDtypeStruct((B, ML, 1), jnp.int32),
        jax.ShapeDtypeStruct((B, ML, 1), jnp.int32),
    ]
    front_out_specs = [
        pl.BlockSpec((1, S, 1), batch3),
        pl.BlockSpec((1, 1, 128), batch3),
        pl.BlockSpec((1, ML, 1), batch3),
        pl.BlockSpec((1, ML, 1), batch3),
        pl.BlockSpec((1, ML, 1), batch3),
    ]
    logdur3, mel3, gidx3, pidx3, eidx3 = pl.pallas_call(
        _front_body,
        grid=(B,),
        in_specs=front_in_specs,
        out_specs=front_out_specs,
        out_shape=front_out_shapes,
    )(x_features, dur3, pt3, et3, pbins, ebins, *dparams)

    xpad = jnp.pad(x.reshape(B * S, D), ((0, 8), (0, 0)))
    xfpad = jnp.pad(x_features.reshape(B * S, D), ((0, 8), (0, 0)))
    gidx_flat = gidx3.reshape(_NROWS)
    pidx_flat = pidx3.reshape(_NROWS)
    eidx_flat = eidx3.reshape(_NROWS)

    mesh = plsc.VectorSubcoreMesh(core_axis_name="c", subcore_axis_name="s")
    sc_call = functools.partial(
        pl.kernel,
        out_type=[
            jax.ShapeDtypeStruct((_NROWS, D), jnp.float32),
            jax.ShapeDtypeStruct((_NROWS, D), jnp.float32),
        ],
        mesh=mesh,
        scratch_types=[
            pltpu.VMEM((_CH,), jnp.int32),
            pltpu.VMEM((_CH,), jnp.int32),
            pltpu.VMEM((_CH,), jnp.int32),
            pltpu.VMEM((_CH, D), jnp.float32),
            pltpu.VMEM((_CH, D), jnp.float32),
            pltpu.VMEM((_CH, D), jnp.float32),
            pltpu.VMEM((_CH, D), jnp.float32),
            pltpu.SemaphoreType.DMA,
            pltpu.SemaphoreType.DMA,
            pltpu.SemaphoreType.DMA,
            pltpu.SemaphoreType.DMA,
        ],
    )(_sc_body)
    xout_flat, xfexp_flat = sc_call(
        xpad, xfpad, pitch_embedding, energy_embedding,
        gidx_flat, pidx_flat, eidx_flat)

    xfexp = xfexp_flat.reshape(B, ML, D)
    back_in_specs = [pl.BlockSpec((1, ML, D), batch3)]
    back_in_specs += [_full_spec(a) for a in pparams + eparams]
    back_out_shapes = [
        jax.ShapeDtypeStruct((B, ML, 1), jnp.float32),
        jax.ShapeDtypeStruct((B, ML, 1), jnp.float32),
    ]
    back_out_specs = [
        pl.BlockSpec((1, ML, 1), batch3),
        pl.BlockSpec((1, ML, 1), batch3),
    ]
    ppred3, epred3 = pl.pallas_call(
        _back_body,
        grid=(B,),
        in_specs=back_in_specs,
        out_specs=back_out_specs,
        out_shape=back_out_shapes,
    )(xfexp, *pparams, *eparams)

    x_out = xout_flat.reshape(B, ML, D)
    log_duration_prediction = logdur3.reshape(B, S)
    pitch_prediction = ppred3.reshape(B, ML)
    energy_prediction = epred3.reshape(B, ML)
    mel_len = mel3[:, 0, 127].astype(jnp.int32)

    return (x_out, log_duration_prediction, duration_target, pitch_prediction,
            energy_prediction, mel_len, mel_mask)
